# baseline (speedup 1.0000x reference)
import functools

import jax
import jax.numpy as jnp
from jax import lax
from jax.experimental import pallas as pl
from jax.experimental.pallas import tpu as pltpu
from jax.experimental.pallas import tpu_sc as plsc

_B = 4096
_N = 100
_R = 128
_BIG = 1e30
_NW = 32
_RPW = _B // _NW
_L = 16


def _sc_body(x_hbm, t_hbm, out_hbm, x_v, t_v, o_v):
    wid = lax.axis_index("s") * 2 + lax.axis_index("c")
    base = wid * _RPW
    pltpu.sync_copy(x_hbm.at[pl.ds(base, _RPW)], x_v)
    pltpu.sync_copy(t_hbm.at[pl.ds(base, _RPW)], t_v)
    lane = lax.iota(jnp.int32, _L)

    def row_step(r, s):
        xv = x_v[r, pl.ds(0, _L)]
        return s + xv[0]

    total = lax.fori_loop(0, _RPW, row_step, jnp.float32(0.0))
    o_v[...] = jnp.where(lane < 0, total, 0.0)
    pltpu.sync_copy(o_v, out_hbm.at[wid])


def _tc_body(x_ref, t_ref, out_ref, acc_ref):
    i = pl.program_id(0)
    x = x_ref[...]
    t = t_ref[...]
    is_pos = t == 1
    pos = is_pos.astype(jnp.float32)
    n_pos = jnp.sum(pos, axis=0, keepdims=True)
    n_neg = (1.0 * _N) - n_pos
    relu = lambda v: jnp.maximum(v, 0.0)
    a = jnp.where(is_pos, -_BIG, 1.0 + x)
    b = jnp.where(is_pos, 1.0 - x, -_BIG)
    neg_sum = jnp.sum(relu(a), axis=0, keepdims=True)
    pos_sum = jnp.sum(relu(b), axis=0, keepdims=True)
    neg_calib_raw = neg_sum / jnp.maximum(n_neg, 1.0)
    pos_calib_raw = pos_sum / jnp.maximum(n_pos, 1.0)
    neg_calib = jnp.where(n_neg > 0, neg_calib_raw, 0.0)
    pos_calib = jnp.where(n_pos > 0, pos_calib_raw, 0.0)
    xb = x.astype(jnp.bfloat16)
    ab = jnp.where(is_pos, jnp.bfloat16(-_BIG), jnp.bfloat16(1.0) + xb)
    acc0 = relu(ab[0:1, :] - xb)
    acc1 = relu(ab[1:2, :] - xb)
    for j in range(2, _N, 2):
        acc0 = acc0 + relu(ab[j : j + 1, :] - xb)
        acc1 = acc1 + relu(ab[j + 1 : j + 2, :] - xb)
    pair_sum = jnp.sum(
        (acc0.astype(jnp.float32) + acc1.astype(jnp.float32)) * pos,
        axis=0, keepdims=True)
    n_pairs = n_neg * n_pos
    pair_mean = pair_sum / jnp.maximum(n_pairs, 1.0)
    l_hinge = jnp.where(
        n_pairs > 0, pair_mean,
        jnp.where((n_neg == 0) & (n_pos == 0), 1.0,
                  jnp.where(n_neg == 0, pos_calib_raw, neg_calib_raw)))
    part = jnp.sum(l_hinge + neg_calib + pos_calib)

    @pl.when(i == 0)
    def _init():
        acc_ref[0] = 0.0

    acc_ref[0] += part

    @pl.when(i == pl.num_programs(0) - 1)
    def _fin():
        out_ref[0] = acc_ref[0] * (1.0 / _B)


@jax.jit
def kernel(outputs, targets):
    sck = functools.partial(
        pl.kernel,
        mesh=plsc.VectorSubcoreMesh(core_axis_name="c", subcore_axis_name="s"),
        out_type=jax.ShapeDtypeStruct((_NW, _L), jnp.float32),
        scratch_types=[
            pltpu.VMEM((_RPW, _N), jnp.float32),
            pltpu.VMEM((_RPW, _N), jnp.int32),
            pltpu.VMEM((_L,), jnp.float32),
        ],
    )(_sc_body)
    partials = sck(outputs, targets)

    xt = outputs.T
    tt = targets.T
    out = pl.pallas_call(
        _tc_body,
        grid=(_B // _R,),
        in_specs=[
            pl.BlockSpec((_N, _R), lambda i: (0, i)),
            pl.BlockSpec((_N, _R), lambda i: (0, i)),
        ],
        out_specs=pl.BlockSpec(memory_space=pltpu.SMEM),
        out_shape=jax.ShapeDtypeStruct((1,), jnp.float32),
        scratch_shapes=[pltpu.SMEM((1,), jnp.float32)],
    )(xt, tt)
    return out[0] + jnp.sum(partials)


# speedup vs baseline: 1.9923x; 1.9923x over previous
import jax
import jax.numpy as jnp
from jax.experimental import pallas as pl
from jax.experimental.pallas import tpu as pltpu

_B = 4096
_N = 100
_R = 128
_BIG = 1e30


def _body(x_ref, t_ref, out_ref, acc_ref):
    i = pl.program_id(0)
    x = x_ref[...]
    t = t_ref[...]
    is_pos = t == 1
    pos = is_pos.astype(jnp.float32)
    n_pos = jnp.sum(pos, axis=0, keepdims=True)
    n_neg = (1.0 * _N) - n_pos

    relu = lambda v: jnp.maximum(v, 0.0)
    xb = x.astype(jnp.bfloat16)
    ab = jnp.where(is_pos, jnp.bfloat16(-_BIG), jnp.bfloat16(1.0) + xb)
    bb = jnp.where(is_pos, jnp.bfloat16(1.0) - xb, jnp.bfloat16(-_BIG))
    neg_sum = jnp.sum(relu(ab).astype(jnp.float32), axis=0, keepdims=True)
    pos_sum = jnp.sum(relu(bb).astype(jnp.float32), axis=0, keepdims=True)
    neg_calib_raw = neg_sum / jnp.maximum(n_neg, 1.0)
    pos_calib_raw = pos_sum / jnp.maximum(n_pos, 1.0)
    neg_calib = jnp.where(n_neg > 0, neg_calib_raw, 0.0)
    pos_calib = jnp.where(n_pos > 0, pos_calib_raw, 0.0)

    acc0 = relu(ab[0:1, :] - xb)
    acc1 = relu(ab[1:2, :] - xb)
    for j in range(2, _N, 2):
        acc0 = acc0 + relu(ab[j : j + 1, :] - xb)
        acc1 = acc1 + relu(ab[j + 1 : j + 2, :] - xb)
    pair_sum = jnp.sum(
        (acc0.astype(jnp.float32) + acc1.astype(jnp.float32)) * pos,
        axis=0,
        keepdims=True,
    )

    n_pairs = n_neg * n_pos
    pair_mean = pair_sum / jnp.maximum(n_pairs, 1.0)
    l_hinge = jnp.where(
        n_pairs > 0,
        pair_mean,
        jnp.where(
            (n_neg == 0) & (n_pos == 0),
            1.0,
            jnp.where(n_neg == 0, pos_calib_raw, neg_calib_raw),
        ),
    )
    part = jnp.sum(l_hinge + neg_calib + pos_calib)

    @pl.when(i == 0)
    def _init():
        acc_ref[0] = 0.0

    acc_ref[0] += part

    @pl.when(i == pl.num_programs(0) - 1)
    def _fin():
        out_ref[0] = acc_ref[0] * (1.0 / _B)


@jax.jit
def kernel(outputs, targets):
    xt = outputs.T
    tt = targets.T
    out = pl.pallas_call(
        _body,
        grid=(_B // _R,),
        in_specs=[
            pl.BlockSpec((_N, _R), lambda i: (0, i)),
            pl.BlockSpec((_N, _R), lambda i: (0, i)),
        ],
        out_specs=pl.BlockSpec(memory_space=pltpu.SMEM),
        out_shape=jax.ShapeDtypeStruct((1,), jnp.float32),
        scratch_shapes=[pltpu.SMEM((1,), jnp.float32)],
    )(xt, tt)
    return out[0]
